# baseline (speedup 1.0000x reference)
import functools

import jax
import jax.numpy as jnp
from jax import lax
from jax.experimental import pallas as pl
from jax.experimental.pallas import tpu as pltpu
from jax.experimental.pallas import tpu_sc as plsc

_P = 4
_H = 96
_W = 96
_B = 64
_C = 96
_NTC = 38
_NSC = _B - _NTC
_NWORKERS = 32
_IMGS_PER_W = _NSC * _C // _NWORKERS
_CB = 32


def _grid_factors(n):
    half = jnp.float32(0.5)
    zero = jnp.float32(0.0)
    gr = (n % (_P * _P)) // _P
    gc = n % _P
    ftop = jnp.where(gr == 0, zero, half)
    fbot = jnp.where(gr == _P - 1, zero, half)
    flft = jnp.where(gc == 0, zero, half)
    frgt = jnp.where(gc == _P - 1, zero, half)
    return ftop, fbot, flft, frgt


def _assemble(ibuf, obuf, ftop, fbot, flft, frgt):
    iota = lax.iota(jnp.int32, 16)

    def ld(rows, cols):
        return plsc.load_gather(ibuf, [rows, cols])

    def st(rows, cols, v, mask=None):
        plsc.store_scatter(obuf, [rows, cols], v, mask=mask)

    c_of = [jnp.full((16,), v, jnp.int32) for v in (0, 1, 94, 95)]

    def row(r, carry):
        rv = jnp.full((16,), r, jnp.int32)
        for t in range(6):
            cols = iota + 16 * t
            st(rv + 1, cols + 1, ld(rv, cols))
        return carry

    lax.fori_loop(0, _H, row, 0, unroll=2)

    z = jnp.full((16,), 0, jnp.int32)
    for t in range(6):
        cols = iota + 16 * t
        st(z, cols + 1, (ld(z, cols) + ld(z + 1, cols)) * ftop)
        st(z + 97, cols + 1, (ld(z + 94, cols) + ld(z + 95, cols)) * fbot)

    for t in range(6):
        rows = iota + 16 * t
        st(rows + 1, z, (ld(rows, c_of[0]) + ld(rows, c_of[1])) * flft)
        st(rows + 1, z + 97, (ld(rows, c_of[2]) + ld(rows, c_of[3])) * frgt)

    ztl = (ftop * 2.0) * (flft * 2.0)
    ztr = (ftop * 2.0) * (frgt * 2.0)
    zbl = (fbot * 2.0) * (flft * 2.0)
    zbr = (fbot * 2.0) * (frgt * 2.0)
    rsrc = jnp.where(iota < 2, 0, 95)
    csrc = jnp.where(iota % 2 == 0, 0, 95)
    rdst = jnp.where(iota < 2, 0, 97)
    cdst = jnp.where(iota % 2 == 0, 0, 97)
    cf = jnp.where(iota == 0, ztl,
                   jnp.where(iota == 1, ztr,
                             jnp.where(iota == 2, zbl, zbr)))
    st(rdst, cdst, ld(rsrc, csrc) * cf, mask=iota < 4)


def _sc_body(x_hbm, out_hbm, ibuf0, ibuf1, obuf0, obuf1,
             sem_i0, sem_i1, sem_o0, sem_o1):
    wid = lax.axis_index("s") * 2 + lax.axis_index("c")
    base = _NTC * _C + wid * _IMGS_PER_W
    ibuf = (ibuf0, ibuf1)
    obuf = (obuf0, obuf1)
    sem_i = (sem_i0, sem_i1)
    sem_o = (sem_o0, sem_o1)

    def start_in(j, b):
        img = base + j
        pltpu.make_async_copy(
            x_hbm.at[img // _C, img % _C], ibuf[b], sem_i[b]).start()

    def wait_in(b):
        pltpu.make_async_copy(
            x_hbm.at[0, 0], ibuf[b], sem_i[b]).wait()

    def start_out(j, b):
        img = base + j
        pltpu.make_async_copy(
            obuf[b], out_hbm.at[img // _C - _NTC, img % _C],
            sem_o[b]).start()

    def wait_out(b):
        pltpu.make_async_copy(
            obuf[b], out_hbm.at[0, 0], sem_o[b]).wait()

    start_in(0, 0)

    def step2(g, carry):
        for b in range(2):
            j = 2 * g + b

            @pl.when(j + 1 < _IMGS_PER_W)
            def _():
                start_in(j + 1, 1 - b)

            @pl.when(j >= 2)
            def _():
                wait_out(b)

            wait_in(b)
            n = (base + j) // _C
            _assemble(ibuf[b], obuf[b], *_grid_factors(n))
            start_out(j, b)
        return carry

    lax.fori_loop(0, _IMGS_PER_W // 2, step2, 0)
    wait_out(0)
    wait_out(1)


@functools.partial(
    pl.kernel,
    out_type=jax.ShapeDtypeStruct((_NSC, _C, _H + 2, _W + 2), jnp.float32),
    mesh=plsc.VectorSubcoreMesh(core_axis_name="c", subcore_axis_name="s"),
    compiler_params=pltpu.CompilerParams(
        needs_layout_passes=False, use_tc_tiling_on_sc=True),
    scratch_types=[
        pltpu.VMEM((_H, _W), jnp.float32),
        pltpu.VMEM((_H, _W), jnp.float32),
        pltpu.VMEM((_H + 2, _W + 2), jnp.float32),
        pltpu.VMEM((_H + 2, _W + 2), jnp.float32),
        pltpu.SemaphoreType.DMA,
        pltpu.SemaphoreType.DMA,
        pltpu.SemaphoreType.DMA,
        pltpu.SemaphoreType.DMA,
    ],
)
def _sc_pad(x_hbm, out_hbm, ibuf0, ibuf1, obuf0, obuf1,
            sem_i0, sem_i1, sem_o0, sem_o1):
    _sc_body(x_hbm, out_hbm, ibuf0, ibuf1, obuf0, obuf1,
             sem_i0, sem_i1, sem_o0, sem_o1)


def _tc_body(x_ref, o_ref):
    n = pl.program_id(0)
    ftop, fbot, flft, frgt = _grid_factors(n)
    xb = x_ref[0]
    lcol = (xb[:, :, 0] + xb[:, :, 1]) * flft
    rcol = (xb[:, :, _W - 2] + xb[:, :, _W - 1]) * frgt
    mid = jnp.concatenate(
        [lcol[:, :, None], xb, rcol[:, :, None]], axis=2)
    o_ref[0, :, 1:_H + 1, :] = mid
    ztl = (ftop * 2.0) * (flft * 2.0)
    ztr = (ftop * 2.0) * (frgt * 2.0)
    zbl = (fbot * 2.0) * (flft * 2.0)
    zbr = (fbot * 2.0) * (frgt * 2.0)
    toprow = jnp.concatenate(
        [(xb[:, 0, 0] * ztl)[:, None],
         (xb[:, 0, :] + xb[:, 1, :]) * ftop,
         (xb[:, 0, _W - 1] * ztr)[:, None]], axis=1)
    botrow = jnp.concatenate(
        [(xb[:, _H - 1, 0] * zbl)[:, None],
         (xb[:, _H - 2, :] + xb[:, _H - 1, :]) * fbot,
         (xb[:, _H - 1, _W - 1] * zbr)[:, None]], axis=1)
    o_ref[0, :, 0, :] = toprow
    o_ref[0, :, _H + 1, :] = botrow


def _tc_pad(x):
    return pl.pallas_call(
        _tc_body,
        grid=(_NTC, _C // _CB),
        in_specs=[pl.BlockSpec((1, _CB, _H, _W), lambda n, c: (n, c, 0, 0))],
        out_specs=pl.BlockSpec((1, _CB, _H + 2, _W + 2),
                               lambda n, c: (n, c, 0, 0)),
        out_shape=jax.ShapeDtypeStruct((_NTC, _C, _H + 2, _W + 2),
                                       jnp.float32),
        compiler_params=pltpu.CompilerParams(
            dimension_semantics=("parallel", "parallel")),
    )(x)


def kernel(x):
    sc_out = _sc_pad(x)
    tc_out = _tc_pad(x)
    return jnp.concatenate([tc_out, sc_out], axis=0)


# speedup vs baseline: 1.2227x; 1.2227x over previous
import functools

import jax
import jax.numpy as jnp
from jax import lax
from jax.experimental import pallas as pl
from jax.experimental.pallas import tpu as pltpu
from jax.experimental.pallas import tpu_sc as plsc

_P = 4
_H = 96
_W = 96
_B = 64
_C = 96
_NTC = 38
_NSC = _B - _NTC
_NWORKERS = 32
_IMGS_PER_W = _NSC * _C // _NWORKERS
_CB = 32


def _grid_factors(n):
    half = jnp.float32(0.5)
    zero = jnp.float32(0.0)
    gr = (n % (_P * _P)) // _P
    gc = n % _P
    ftop = jnp.where(gr == 0, zero, half)
    fbot = jnp.where(gr == _P - 1, zero, half)
    flft = jnp.where(gc == 0, zero, half)
    frgt = jnp.where(gc == _P - 1, zero, half)
    return ftop, fbot, flft, frgt


def _assemble(ibuf, obuf, ftop, fbot, flft, frgt):
    iota = lax.iota(jnp.int32, 16)

    def ld(rows, cols):
        return plsc.load_gather(ibuf, [rows, cols])

    def st(rows, cols, v, mask=None):
        plsc.store_scatter(obuf, [rows, cols], v, mask=mask)

    c_of = [jnp.full((16,), v, jnp.int32) for v in (0, 1, 94, 95)]

    def row(r, carry):
        rv = jnp.full((16,), r, jnp.int32)
        for t in range(6):
            cols = iota + 16 * t
            st(rv + 1, cols + 1, ld(rv, cols))
        return carry

    lax.fori_loop(0, _H, row, 0, unroll=2)

    z = jnp.full((16,), 0, jnp.int32)
    for t in range(6):
        cols = iota + 16 * t
        st(z, cols + 1, (ld(z, cols) + ld(z + 1, cols)) * ftop)
        st(z + 97, cols + 1, (ld(z + 94, cols) + ld(z + 95, cols)) * fbot)

    for t in range(6):
        rows = iota + 16 * t
        st(rows + 1, z, (ld(rows, c_of[0]) + ld(rows, c_of[1])) * flft)
        st(rows + 1, z + 97, (ld(rows, c_of[2]) + ld(rows, c_of[3])) * frgt)

    ztl = (ftop * 2.0) * (flft * 2.0)
    ztr = (ftop * 2.0) * (frgt * 2.0)
    zbl = (fbot * 2.0) * (flft * 2.0)
    zbr = (fbot * 2.0) * (frgt * 2.0)
    rsrc = jnp.where(iota < 2, 0, 95)
    csrc = jnp.where(iota % 2 == 0, 0, 95)
    rdst = jnp.where(iota < 2, 0, 97)
    cdst = jnp.where(iota % 2 == 0, 0, 97)
    cf = jnp.where(iota == 0, ztl,
                   jnp.where(iota == 1, ztr,
                             jnp.where(iota == 2, zbl, zbr)))
    st(rdst, cdst, ld(rsrc, csrc) * cf, mask=iota < 4)


def _sc_body(x_hbm, out_hbm, ibuf0, ibuf1, obuf0, obuf1,
             sem_i0, sem_i1, sem_o0, sem_o1):
    wid = lax.axis_index("s") * 2 + lax.axis_index("c")
    base = _NTC * _C + wid * _IMGS_PER_W
    ibuf = (ibuf0, ibuf1)
    obuf = (obuf0, obuf1)
    sem_i = (sem_i0, sem_i1)
    sem_o = (sem_o0, sem_o1)

    def start_in(j, b):
        img = base + j
        pltpu.make_async_copy(
            x_hbm.at[img // _C, img % _C], ibuf[b], sem_i[b]).start()

    def wait_in(b):
        pltpu.make_async_copy(
            x_hbm.at[0, 0], ibuf[b], sem_i[b]).wait()

    def start_out(j, b):
        img = base + j
        pltpu.make_async_copy(
            obuf[b], out_hbm.at[img // _C - _NTC, img % _C],
            sem_o[b]).start()

    def wait_out(b):
        pltpu.make_async_copy(
            obuf[b], out_hbm.at[0, 0], sem_o[b]).wait()

    start_in(0, 0)

    def step2(g, carry):
        for b in range(2):
            j = 2 * g + b

            @pl.when(j + 1 < _IMGS_PER_W)
            def _():
                start_in(j + 1, 1 - b)

            @pl.when(j >= 2)
            def _():
                wait_out(b)

            wait_in(b)
            n = (base + j) // _C
            _assemble(ibuf[b], obuf[b], *_grid_factors(n))
            start_out(j, b)
        return carry

    lax.fori_loop(0, _IMGS_PER_W // 2, step2, 0)
    wait_out(0)
    wait_out(1)


@functools.partial(
    pl.kernel,
    out_type=jax.ShapeDtypeStruct((_NSC, _C, _H + 2, _W + 2), jnp.float32),
    mesh=plsc.VectorSubcoreMesh(core_axis_name="c", subcore_axis_name="s"),
    compiler_params=pltpu.CompilerParams(
        needs_layout_passes=False, use_tc_tiling_on_sc=True),
    scratch_types=[
        pltpu.VMEM((_H, _W), jnp.float32),
        pltpu.VMEM((_H, _W), jnp.float32),
        pltpu.VMEM((_H + 2, _W + 2), jnp.float32),
        pltpu.VMEM((_H + 2, _W + 2), jnp.float32),
        pltpu.SemaphoreType.DMA,
        pltpu.SemaphoreType.DMA,
        pltpu.SemaphoreType.DMA,
        pltpu.SemaphoreType.DMA,
    ],
)
def _sc_pad(x_hbm, out_hbm, ibuf0, ibuf1, obuf0, obuf1,
            sem_i0, sem_i1, sem_o0, sem_o1):
    _sc_body(x_hbm, out_hbm, ibuf0, ibuf1, obuf0, obuf1,
             sem_i0, sem_i1, sem_o0, sem_o1)


def _tc_body(x_ref, o_ref):
    n = pl.program_id(0)
    ftop, fbot, flft, frgt = _grid_factors(n)
    xb = x_ref[0]
    lcol = (xb[:, :, 0] + xb[:, :, 1]) * flft
    rcol = (xb[:, :, _W - 2] + xb[:, :, _W - 1]) * frgt
    mid = jnp.concatenate(
        [lcol[:, :, None], xb, rcol[:, :, None]], axis=2)
    o_ref[0, :, 1:_H + 1, :] = mid
    ztl = (ftop * 2.0) * (flft * 2.0)
    ztr = (ftop * 2.0) * (frgt * 2.0)
    zbl = (fbot * 2.0) * (flft * 2.0)
    zbr = (fbot * 2.0) * (frgt * 2.0)
    toprow = jnp.concatenate(
        [(xb[:, 0, 0] * ztl)[:, None],
         (xb[:, 0, :] + xb[:, 1, :]) * ftop,
         (xb[:, 0, _W - 1] * ztr)[:, None]], axis=1)
    botrow = jnp.concatenate(
        [(xb[:, _H - 1, 0] * zbl)[:, None],
         (xb[:, _H - 2, :] + xb[:, _H - 1, :]) * fbot,
         (xb[:, _H - 1, _W - 1] * zbr)[:, None]], axis=1)
    o_ref[0, :, 0, :] = toprow
    o_ref[0, :, _H + 1, :] = botrow


def _tc_pad(x):
    return pl.pallas_call(
        _tc_body,
        grid=(_NTC, _C // _CB),
        in_specs=[pl.BlockSpec((1, _CB, _H, _W), lambda n, c: (n, c, 0, 0))],
        out_specs=pl.BlockSpec((1, _CB, _H + 2, _W + 2),
                               lambda n, c: (n, c, 0, 0)),
        out_shape=jax.ShapeDtypeStruct((_B, _C, _H + 2, _W + 2),
                                       jnp.float32),
        compiler_params=pltpu.CompilerParams(
            dimension_semantics=("parallel", "parallel")),
    )(x)


def kernel(x):
    sc_out = _sc_pad(x)
    tc_out = _tc_pad(x)
    return lax.dynamic_update_slice(tc_out, sc_out, (_NTC, 0, 0, 0))
